# Initial kernel scaffold; baseline (speedup 1.0000x reference)
import jax
import jax.numpy as jnp
from jax.experimental import pallas as pl


def kernel(e_s, industry_matrix, W, b):
    raise NotImplementedError("write your pallas kernel here")


# speedup vs baseline: 189.3634x; 189.3634x over previous
import jax
import jax.numpy as jnp
from jax.experimental import pallas as pl
from jax.experimental.pallas import tpu as pltpu

_N = 50000
_M = 16
_H = 128
_BLK = 2000
_NBLK = _N // _BLK


def _pass1(e_ref, m_ref, w_ref, he_ref, s_acc, cnt_acc):
    i = pl.program_id(0)

    @pl.when(i == 0)
    def _init():
        s_acc[...] = jnp.zeros_like(s_acc)
        cnt_acc[...] = jnp.zeros_like(cnt_acc)

    maskf = (m_ref[...] == 1).astype(jnp.float32)
    s_acc[...] += jax.lax.dot_general(
        maskf, e_ref[...], (((0,), (0,)), ((), ())),
        preferred_element_type=jnp.float32)
    cnt_acc[...] += jnp.sum(maskf, axis=0)[None, :]

    @pl.when(i == _NBLK - 1)
    def _fini():
        cnt = cnt_acc[0, :]
        binv = jnp.where(cnt > 0, 1.0 / cnt, 0.0)
        sw = jax.lax.dot_general(
            s_acc[...], w_ref[...], (((1,), (1,)), ((), ())),
            preferred_element_type=jnp.float32)
        he_ref[...] = binv[:, None] * sw


def _pass2(m_ref, he_ref, b_ref, out_ref):
    maskf = (m_ref[...] == 1).astype(jnp.float32)
    d = jnp.sum(maskf, axis=1)
    dinv = jnp.where(d > 0, 1.0 / d, 0.0)
    y = jnp.dot(maskf, he_ref[...], preferred_element_type=jnp.float32)
    y = y * dinv[:, None] + b_ref[...]
    out_ref[...] = jnp.where(y >= 0, y, 0.01 * y)


def kernel(e_s, industry_matrix, W, b):
    he = pl.pallas_call(
        _pass1,
        grid=(_NBLK,),
        in_specs=[
            pl.BlockSpec((_BLK, _H), lambda i: (i, 0)),
            pl.BlockSpec((_BLK, _M), lambda i: (i, 0)),
            pl.BlockSpec((_H, _H), lambda i: (0, 0)),
        ],
        out_specs=pl.BlockSpec((_M, _H), lambda i: (0, 0)),
        out_shape=jax.ShapeDtypeStruct((_M, _H), jnp.float32),
        scratch_shapes=[
            pltpu.VMEM((_M, _H), jnp.float32),
            pltpu.VMEM((1, _M), jnp.float32),
        ],
    )(e_s, industry_matrix, W)

    out = pl.pallas_call(
        _pass2,
        grid=(_NBLK,),
        in_specs=[
            pl.BlockSpec((_BLK, _M), lambda i: (i, 0)),
            pl.BlockSpec((_M, _H), lambda i: (0, 0)),
            pl.BlockSpec((1, _H), lambda i: (0, 0)),
        ],
        out_specs=pl.BlockSpec((_BLK, _H), lambda i: (i, 0)),
        out_shape=jax.ShapeDtypeStruct((_N, _H), jnp.float32),
    )(industry_matrix, he, b.reshape(1, _H))
    return out


# speedup vs baseline: 246.7240x; 1.3029x over previous
import jax
import jax.numpy as jnp
from jax.experimental import pallas as pl
from jax.experimental.pallas import tpu as pltpu

_N = 50000
_M = 16
_H = 128
_BLK = 5000
_NBLK = _N // _BLK


def _pass1(e_ref, m_ref, w_ref, he_ref, s_acc, cnt_acc):
    i = pl.program_id(0)

    @pl.when(i == 0)
    def _init():
        s_acc[...] = jnp.zeros_like(s_acc)
        cnt_acc[...] = jnp.zeros_like(cnt_acc)

    maskf = (m_ref[...] == 1).astype(jnp.float32)
    s_acc[...] += jax.lax.dot_general(
        maskf, e_ref[...], (((0,), (0,)), ((), ())),
        preferred_element_type=jnp.float32)
    cnt_acc[...] += jnp.sum(maskf, axis=0)[None, :]

    @pl.when(i == _NBLK - 1)
    def _fini():
        cnt = cnt_acc[0, :]
        binv = jnp.where(cnt > 0, 1.0 / cnt, 0.0)
        sw = jax.lax.dot_general(
            s_acc[...], w_ref[...], (((1,), (1,)), ((), ())),
            preferred_element_type=jnp.float32)
        he_ref[...] = binv[:, None] * sw


def _pass2(m_ref, he_ref, b_ref, out_ref):
    maskf = (m_ref[...] == 1).astype(jnp.float32)
    d = jnp.sum(maskf, axis=1)
    dinv = jnp.where(d > 0, 1.0 / d, 0.0)
    y = jnp.dot(maskf, he_ref[...], preferred_element_type=jnp.float32)
    y = y * dinv[:, None] + b_ref[...]
    out_ref[...] = jnp.where(y >= 0, y, 0.01 * y)


def kernel(e_s, industry_matrix, W, b):
    he = pl.pallas_call(
        _pass1,
        grid=(_NBLK,),
        in_specs=[
            pl.BlockSpec((_BLK, _H), lambda i: (i, 0)),
            pl.BlockSpec((_BLK, _M), lambda i: (i, 0)),
            pl.BlockSpec((_H, _H), lambda i: (0, 0)),
        ],
        out_specs=pl.BlockSpec((_M, _H), lambda i: (0, 0)),
        out_shape=jax.ShapeDtypeStruct((_M, _H), jnp.float32),
        scratch_shapes=[
            pltpu.VMEM((_M, _H), jnp.float32),
            pltpu.VMEM((1, _M), jnp.float32),
        ],
    )(e_s, industry_matrix, W)

    out = pl.pallas_call(
        _pass2,
        grid=(_NBLK,),
        in_specs=[
            pl.BlockSpec((_BLK, _M), lambda i: (i, 0)),
            pl.BlockSpec((_M, _H), lambda i: (0, 0)),
            pl.BlockSpec((1, _H), lambda i: (0, 0)),
        ],
        out_specs=pl.BlockSpec((_BLK, _H), lambda i: (i, 0)),
        out_shape=jax.ShapeDtypeStruct((_N, _H), jnp.float32),
    )(industry_matrix, he, b.reshape(1, _H))
    return out


# speedup vs baseline: 260.0960x; 1.0542x over previous
import jax
import jax.numpy as jnp
from jax.experimental import pallas as pl
from jax.experimental.pallas import tpu as pltpu

_N = 50000
_M = 16
_H = 128
_BLK = 10000
_NBLK = _N // _BLK


def _pass1(e_ref, m_ref, w_ref, he_ref, s_acc, cnt_acc):
    i = pl.program_id(0)

    @pl.when(i == 0)
    def _init():
        s_acc[...] = jnp.zeros_like(s_acc)
        cnt_acc[...] = jnp.zeros_like(cnt_acc)

    maskf = (m_ref[...] == 1).astype(jnp.float32)
    s_acc[...] += jax.lax.dot_general(
        maskf, e_ref[...], (((0,), (0,)), ((), ())),
        preferred_element_type=jnp.float32)
    cnt_acc[...] += jnp.sum(maskf, axis=0)[None, :]

    @pl.when(i == _NBLK - 1)
    def _fini():
        cnt = cnt_acc[0, :]
        binv = jnp.where(cnt > 0, 1.0 / cnt, 0.0)
        sw = jax.lax.dot_general(
            s_acc[...], w_ref[...], (((1,), (1,)), ((), ())),
            preferred_element_type=jnp.float32)
        he_ref[...] = binv[:, None] * sw


def _pass2(m_ref, he_ref, b_ref, out_ref):
    maskf = (m_ref[...] == 1).astype(jnp.float32)
    d = jnp.sum(maskf, axis=1)
    dinv = jnp.where(d > 0, 1.0 / d, 0.0)
    y = jnp.dot(maskf, he_ref[...], preferred_element_type=jnp.float32)
    y = y * dinv[:, None] + b_ref[...]
    out_ref[...] = jnp.where(y >= 0, y, 0.01 * y)


def kernel(e_s, industry_matrix, W, b):
    he = pl.pallas_call(
        _pass1,
        grid=(_NBLK,),
        in_specs=[
            pl.BlockSpec((_BLK, _H), lambda i: (i, 0)),
            pl.BlockSpec((_BLK, _M), lambda i: (i, 0)),
            pl.BlockSpec((_H, _H), lambda i: (0, 0)),
        ],
        out_specs=pl.BlockSpec((_M, _H), lambda i: (0, 0)),
        out_shape=jax.ShapeDtypeStruct((_M, _H), jnp.float32),
        scratch_shapes=[
            pltpu.VMEM((_M, _H), jnp.float32),
            pltpu.VMEM((1, _M), jnp.float32),
        ],
    )(e_s, industry_matrix, W)

    out = pl.pallas_call(
        _pass2,
        grid=(_NBLK,),
        in_specs=[
            pl.BlockSpec((_BLK, _M), lambda i: (i, 0)),
            pl.BlockSpec((_M, _H), lambda i: (0, 0)),
            pl.BlockSpec((1, _H), lambda i: (0, 0)),
        ],
        out_specs=pl.BlockSpec((_BLK, _H), lambda i: (i, 0)),
        out_shape=jax.ShapeDtypeStruct((_N, _H), jnp.float32),
    )(industry_matrix, he, b.reshape(1, _H))
    return out


# speedup vs baseline: 280.2602x; 1.0775x over previous
import jax
import jax.numpy as jnp
from jax.experimental import pallas as pl
from jax.experimental.pallas import tpu as pltpu

_N = 50000
_M = 16
_H = 128
_BLK = 10000
_NB = _N // _BLK


def _fused(e_ref, m_ref, w_ref, b_ref, out_ref, s_acc, cnt_acc, he_s):
    i = pl.program_id(0)

    @pl.when(i == 0)
    def _init():
        s_acc[...] = jnp.zeros_like(s_acc)
        cnt_acc[...] = jnp.zeros_like(cnt_acc)

    @pl.when(i < _NB)
    def _reduce():
        maskf = m_ref[...]
        s_acc[...] += jax.lax.dot_general(
            maskf, e_ref[...], (((0,), (0,)), ((), ())),
            preferred_element_type=jnp.float32)
        cnt_acc[...] += jnp.sum(maskf, axis=0)[None, :]

    @pl.when(i == _NB - 1)
    def _epilogue():
        cnt = cnt_acc[0, :]
        binv = jnp.where(cnt > 0, 1.0 / cnt, 0.0)
        sw = jax.lax.dot_general(
            s_acc[...], w_ref[...], (((1,), (1,)), ((), ())),
            preferred_element_type=jnp.float32)
        he_s[...] = binv[:, None] * sw

    @pl.when(i >= _NB)
    def _emit():
        maskf = m_ref[...]
        d = jnp.sum(maskf, axis=1)
        dinv = jnp.where(d > 0, 1.0 / d, 0.0)
        y = jnp.dot(maskf, he_s[...], preferred_element_type=jnp.float32)
        y = y * dinv[:, None] + b_ref[...]
        out_ref[...] = jnp.where(y >= 0, y, 0.01 * y)


def kernel(e_s, industry_matrix, W, b):
    maskf = industry_matrix.astype(jnp.float32)
    out = pl.pallas_call(
        _fused,
        grid=(2 * _NB,),
        in_specs=[
            pl.BlockSpec((_BLK, _H), lambda i: (jnp.minimum(i, _NB - 1), 0)),
            pl.BlockSpec((_BLK, _M), lambda i: (i % _NB, 0)),
            pl.BlockSpec((_H, _H), lambda i: (0, 0)),
            pl.BlockSpec((1, _H), lambda i: (0, 0)),
        ],
        out_specs=pl.BlockSpec(
            (_BLK, _H), lambda i: (jnp.maximum(i - _NB, 0), 0)),
        out_shape=jax.ShapeDtypeStruct((_N, _H), jnp.float32),
        scratch_shapes=[
            pltpu.VMEM((_M, _H), jnp.float32),
            pltpu.VMEM((1, _M), jnp.float32),
            pltpu.VMEM((_M, _H), jnp.float32),
        ],
    )(e_s, maskf, W, b.reshape(1, _H))
    return out


# speedup vs baseline: 301.4367x; 1.0756x over previous
import jax
import jax.numpy as jnp
from jax.experimental import pallas as pl
from jax.experimental.pallas import tpu as pltpu

_N = 50000
_M = 16
_H = 128
_BLK = 10000
_NB = _N // _BLK


def _fused(e_ref, m_ref, w_ref, b_ref, out_ref, s_acc, cnt_acc, he_s):
    i = pl.program_id(0)

    @pl.when(i == 0)
    def _init():
        s_acc[...] = jnp.zeros_like(s_acc)
        cnt_acc[...] = jnp.zeros_like(cnt_acc)

    @pl.when(i < _NB)
    def _reduce():
        maskf = m_ref[pl.ds(i * _BLK, _BLK), :]
        s_acc[...] += jax.lax.dot_general(
            maskf, e_ref[...], (((0,), (0,)), ((), ())),
            preferred_element_type=jnp.float32)
        cnt_acc[...] += jnp.sum(maskf, axis=0)[None, :]

    @pl.when(i == _NB - 1)
    def _epilogue():
        cnt = cnt_acc[0, :]
        binv = jnp.where(cnt > 0, 1.0 / cnt, 0.0)
        sw = jax.lax.dot_general(
            s_acc[...], w_ref[...], (((1,), (1,)), ((), ())),
            preferred_element_type=jnp.float32)
        he_s[...] = binv[:, None] * sw

    @pl.when(i >= _NB)
    def _emit():
        maskf = m_ref[pl.ds((i - _NB) * _BLK, _BLK), :]
        d = jnp.sum(maskf, axis=1)
        dinv = jnp.where(d > 0, 1.0 / d, 0.0)
        y = jnp.dot(maskf, he_s[...], preferred_element_type=jnp.float32)
        y = y * dinv[:, None] + b_ref[...]
        out_ref[...] = jnp.where(y >= 0, y, 0.01 * y)


def kernel(e_s, industry_matrix, W, b):
    maskf = industry_matrix.astype(jnp.float32)
    out = pl.pallas_call(
        _fused,
        grid=(2 * _NB,),
        in_specs=[
            pl.BlockSpec((_BLK, _H), lambda i: (jnp.minimum(i, _NB - 1), 0)),
            pl.BlockSpec((_N, _M), lambda i: (0, 0)),
            pl.BlockSpec((_H, _H), lambda i: (0, 0)),
            pl.BlockSpec((1, _H), lambda i: (0, 0)),
        ],
        out_specs=pl.BlockSpec(
            (_BLK, _H), lambda i: (jnp.maximum(i - _NB, 0), 0)),
        out_shape=jax.ShapeDtypeStruct((_N, _H), jnp.float32),
        scratch_shapes=[
            pltpu.VMEM((_M, _H), jnp.float32),
            pltpu.VMEM((1, _M), jnp.float32),
            pltpu.VMEM((_M, _H), jnp.float32),
        ],
    )(e_s, maskf, W, b.reshape(1, _H))
    return out
